# Initial kernel scaffold; baseline (speedup 1.0000x reference)
import functools

import jax
import jax.numpy as jnp
from jax import lax
from jax.experimental import pallas as pl
from jax.experimental.pallas import tpu as pltpu
from jax.experimental.pallas import tpu_sc as plsc

N = 10000
E = 320000
D = 128
DA = 144
H = 128
G = 128

_info = plsc.get_sparse_core_info()
NC = _info.num_cores
NS = _info.num_subcores
NW = NC * NS
GPT = -(-E // (G * NW))
E_PAD = GPT * G * NW
ROWS_PT = N // NS
ACC_ROWS = N + 8


def _bn_body(x_ref, w_ref, b_ref, xn_ref, xa_ref):
    x = x_ref[...]
    mu = jnp.mean(x, axis=0, keepdims=True)
    xc = x - mu
    var = jnp.mean(xc * xc, axis=0, keepdims=True)
    xn = xc * lax.rsqrt(var + 1e-5) * w_ref[...] + b_ref[...]
    xn_ref[...] = xn
    xa_ref[...] = jnp.concatenate(
        [xn, jnp.ones((x.shape[0], DA - D), jnp.float32)], axis=1)


_bn_call = pl.pallas_call(
    _bn_body,
    out_shape=(jax.ShapeDtypeStruct((N, D), jnp.float32),
               jax.ShapeDtypeStruct((N, DA), jnp.float32)),
)


_mesh = plsc.VectorSubcoreMesh(core_axis_name="c", subcore_axis_name="s")


@functools.partial(
    pl.kernel,
    out_type=jax.ShapeDtypeStruct((NC, N, DA), jnp.float32),
    mesh=_mesh,
    scratch_types=[
        pltpu.VMEM((G,), jnp.int32),
        pltpu.VMEM((G,), jnp.int32),
        pltpu.VMEM((G, DA), jnp.float32),
        pltpu.VMEM_SHARED((ACC_ROWS, DA), jnp.float32),
        pltpu.SemaphoreType.DMA,
    ],
)
def _sc_scatter(xa_hbm, src_hbm, dst_hbm, zeros_hbm, out_hbm,
                src_v, dst_v, rows_v, acc_sh, sem):
    c = lax.axis_index("c")
    s = lax.axis_index("s")
    pltpu.sync_copy(zeros_hbm.at[pl.ds(s * ROWS_PT, ROWS_PT)],
                    acc_sh.at[pl.ds(s * ROWS_PT, ROWS_PT)])
    plsc.subcore_barrier()

    wid = s * NC + c
    base = wid * GPT

    def body(g, carry):
        row = base + g
        pltpu.sync_copy(src_hbm.at[row], src_v)
        pltpu.sync_copy(dst_hbm.at[row], dst_v)
        pltpu.async_copy(xa_hbm.at[src_v], rows_v, sem).wait()
        pltpu.sync_copy(rows_v, acc_sh.at[dst_v], add=True)
        return carry

    lax.fori_loop(0, GPT, body, 0)
    plsc.subcore_barrier()
    pltpu.sync_copy(acc_sh.at[pl.ds(s * ROWS_PT, ROWS_PT)],
                    out_hbm.at[c].at[pl.ds(s * ROWS_PT, ROWS_PT)])


_BN = 2000

_DN = (((1,), (1,)), ((), ()))


def _head_body(a0_ref, a1_ref, xn_ref, wl_ref, bl_ref, wr_ref,
               wc1_ref, bc1_ref, wc2_ref, bc2_ref, out_ref):
    a = a0_ref[...] + a1_ref[...]
    ssum = a[:, :D]
    cnt = a[:, D:D + 1]
    agg = ssum / jnp.maximum(cnt, 1.0)
    xn = xn_ref[...]
    hp = lax.Precision.HIGHEST
    h = (lax.dot_general(agg, wl_ref[...], _DN, precision=hp)
         + lax.dot_general(xn, wr_ref[...], _DN, precision=hp)
         + bl_ref[...])
    h = jnp.maximum(h, 0.0)
    h1 = jnp.maximum(
        lax.dot_general(h, wc1_ref[...], _DN, precision=hp) + bc1_ref[...],
        0.0)
    out_ref[...] = (lax.dot_general(h1, wc2_ref[...], _DN, precision=hp)
                    + bc2_ref[...])


_head_call = pl.pallas_call(
    _head_body,
    grid=(N // _BN,),
    in_specs=[
        pl.BlockSpec((_BN, DA), lambda i: (i, 0)),
        pl.BlockSpec((_BN, DA), lambda i: (i, 0)),
        pl.BlockSpec((_BN, D), lambda i: (i, 0)),
        pl.BlockSpec((H, D), lambda i: (0, 0)),
        pl.BlockSpec((1, H), lambda i: (0, 0)),
        pl.BlockSpec((H, D), lambda i: (0, 0)),
        pl.BlockSpec((16, H), lambda i: (0, 0)),
        pl.BlockSpec((1, 16), lambda i: (0, 0)),
        pl.BlockSpec((2, 16), lambda i: (0, 0)),
        pl.BlockSpec((1, 2), lambda i: (0, 0)),
    ],
    out_specs=pl.BlockSpec((_BN, 2), lambda i: (i, 0)),
    out_shape=jax.ShapeDtypeStruct((N, 2), jnp.float32),
)


def kernel(x, edge_index, edge_weight, edge_features, adj, T,
           bn_weight, bn_bias, W_l, b_l, W_r, Wc1, bc1, Wc2, bc2):
    xn, xa = _bn_call(x, bn_weight.reshape(1, D), bn_bias.reshape(1, D))

    pad = E_PAD - E
    src = jnp.concatenate([edge_index[0], jnp.zeros((pad,), jnp.int32)])
    dst = jnp.concatenate([edge_index[1], jnp.full((pad,), N, jnp.int32)])
    src2 = src.reshape(NW * GPT, G)
    dst2 = dst.reshape(NW * GPT, G)
    zeros = jnp.zeros((N, DA), jnp.float32)

    acc = _sc_scatter(xa, src2, dst2, zeros)

    out = _head_call(acc[0], acc[1], xn, W_l, b_l.reshape(1, H), W_r,
                     Wc1, bc1.reshape(1, 16), Wc2, bc2.reshape(1, 2))
    return out


# speedup vs baseline: 4.3157x; 4.3157x over previous
import functools

import jax
import jax.numpy as jnp
from jax import lax
from jax.experimental import pallas as pl
from jax.experimental.pallas import tpu as pltpu
from jax.experimental.pallas import tpu_sc as plsc

N = 10000
E = 320000
D = 128
DA = 144
H = 128
G = 128

_info = plsc.get_sparse_core_info()
NC = _info.num_cores
NS = _info.num_subcores
NW = NC * NS
GPT = -(-E // (G * NW))
E_PAD = GPT * G * NW
ROWS_PT = 632
ACC_ROWS = ROWS_PT * NS


def _bn_body(x_ref, w_ref, b_ref, xn_ref, xa_ref):
    x = x_ref[...]
    mu = jnp.mean(x, axis=0, keepdims=True)
    xc = x - mu
    var = jnp.mean(xc * xc, axis=0, keepdims=True)
    xn = xc * lax.rsqrt(var + 1e-5) * w_ref[...] + b_ref[...]
    xn_ref[...] = xn
    xa_ref[...] = jnp.concatenate(
        [xn, jnp.ones((x.shape[0], DA - D), jnp.float32)], axis=1)


_bn_call = pl.pallas_call(
    _bn_body,
    out_shape=(jax.ShapeDtypeStruct((N, D), jnp.float32),
               jax.ShapeDtypeStruct((N, DA), jnp.float32)),
)


_mesh = plsc.VectorSubcoreMesh(core_axis_name="c", subcore_axis_name="s")


@functools.partial(
    pl.kernel,
    out_type=jax.ShapeDtypeStruct((NC, ACC_ROWS, DA), jnp.float32),
    mesh=_mesh,
    scratch_types=[
        pltpu.VMEM((G,), jnp.int32),
        pltpu.VMEM((G,), jnp.int32),
        pltpu.VMEM((G, DA), jnp.float32),
        pltpu.VMEM_SHARED((ACC_ROWS, DA), jnp.float32),
        pltpu.SemaphoreType.DMA,
    ],
    compiler_params=pltpu.CompilerParams(use_tc_tiling_on_sc=False),
)
def _sc_scatter(xa_hbm, src_hbm, dst_hbm, zeros_hbm, out_hbm,
                src_v, dst_v, rows_v, acc_sh, sem):
    c = lax.axis_index("c")
    s = lax.axis_index("s")
    pltpu.sync_copy(zeros_hbm.at[pl.ds(s * ROWS_PT, ROWS_PT)],
                    acc_sh.at[pl.ds(s * ROWS_PT, ROWS_PT)])
    plsc.subcore_barrier()

    wid = s * NC + c
    base = wid * GPT

    def body(g, carry):
        row = base + g
        pltpu.sync_copy(src_hbm.at[row], src_v)
        pltpu.sync_copy(dst_hbm.at[row], dst_v)
        pltpu.async_copy(xa_hbm.at[src_v], rows_v, sem).wait()
        pltpu.sync_copy(rows_v, acc_sh.at[dst_v], add=True)
        return carry

    lax.fori_loop(0, GPT, body, 0)
    plsc.subcore_barrier()
    pltpu.sync_copy(acc_sh.at[pl.ds(s * ROWS_PT, ROWS_PT)],
                    out_hbm.at[c].at[pl.ds(s * ROWS_PT, ROWS_PT)])


_BN = 2000

_DN = (((1,), (1,)), ((), ()))


def _head_body(a0_ref, a1_ref, xn_ref, wl_ref, bl_ref, wr_ref,
               wc1_ref, bc1_ref, wc2_ref, bc2_ref, out_ref):
    a = a0_ref[...] + a1_ref[...]
    ssum = a[:, :D]
    cnt = a[:, D:D + 1]
    agg = ssum / jnp.maximum(cnt, 1.0)
    xn = xn_ref[...]
    hp = lax.Precision.HIGHEST
    h = (lax.dot_general(agg, wl_ref[...], _DN, precision=hp)
         + lax.dot_general(xn, wr_ref[...], _DN, precision=hp)
         + bl_ref[...])
    h = jnp.maximum(h, 0.0)
    h1 = jnp.maximum(
        lax.dot_general(h, wc1_ref[...], _DN, precision=hp) + bc1_ref[...],
        0.0)
    out_ref[...] = (lax.dot_general(h1, wc2_ref[...], _DN, precision=hp)
                    + bc2_ref[...])


_head_call = pl.pallas_call(
    _head_body,
    grid=(N // _BN,),
    in_specs=[
        pl.BlockSpec((_BN, DA), lambda i: (i, 0)),
        pl.BlockSpec((_BN, DA), lambda i: (i, 0)),
        pl.BlockSpec((_BN, D), lambda i: (i, 0)),
        pl.BlockSpec((H, D), lambda i: (0, 0)),
        pl.BlockSpec((1, H), lambda i: (0, 0)),
        pl.BlockSpec((H, D), lambda i: (0, 0)),
        pl.BlockSpec((16, H), lambda i: (0, 0)),
        pl.BlockSpec((1, 16), lambda i: (0, 0)),
        pl.BlockSpec((2, 16), lambda i: (0, 0)),
        pl.BlockSpec((1, 2), lambda i: (0, 0)),
    ],
    out_specs=pl.BlockSpec((_BN, 2), lambda i: (i, 0)),
    out_shape=jax.ShapeDtypeStruct((N, 2), jnp.float32),
)


def kernel(x, edge_index, edge_weight, edge_features, adj, T,
           bn_weight, bn_bias, W_l, b_l, W_r, Wc1, bc1, Wc2, bc2):
    xn, xa = _bn_call(x, bn_weight.reshape(1, D), bn_bias.reshape(1, D))

    pad = E_PAD - E
    src = jnp.concatenate([edge_index[0], jnp.zeros((pad,), jnp.int32)])
    dst = jnp.concatenate([edge_index[1], jnp.full((pad,), N, jnp.int32)])
    src2 = src.reshape(NW * GPT, G)
    dst2 = dst.reshape(NW * GPT, G)
    zeros = jnp.zeros((ACC_ROWS, DA), jnp.float32)

    acc = _sc_scatter(xa, src2, dst2, zeros)

    out = _head_call(acc[0], acc[1], xn, W_l, b_l.reshape(1, H), W_r,
                     Wc1, bc1.reshape(1, 16), Wc2, bc2.reshape(1, 2))
    return out


# speedup vs baseline: 5.3757x; 1.2456x over previous
import functools

import jax
import jax.numpy as jnp
from jax import lax
from jax.experimental import pallas as pl
from jax.experimental.pallas import tpu as pltpu
from jax.experimental.pallas import tpu_sc as plsc

N = 10000
E = 320000
D = 128
DA = 144
H = 128
G = 128

_info = plsc.get_sparse_core_info()
NC = _info.num_cores
NS = _info.num_subcores
NW = NC * NS
GPT = -(-E // (G * NW))
E_PAD = GPT * G * NW
ROWS_PT = 632
ACC_ROWS = ROWS_PT * NS


def _bn_body(x_ref, w_ref, b_ref, xn_ref, xa_ref):
    x = x_ref[...]
    mu = jnp.mean(x, axis=0, keepdims=True)
    xc = x - mu
    var = jnp.mean(xc * xc, axis=0, keepdims=True)
    xn = xc * lax.rsqrt(var + 1e-5) * w_ref[...] + b_ref[...]
    xn_ref[...] = xn
    xa_ref[...] = jnp.concatenate(
        [xn, jnp.ones((x.shape[0], DA - D), jnp.float32)], axis=1)


_bn_call = pl.pallas_call(
    _bn_body,
    out_shape=(jax.ShapeDtypeStruct((N, D), jnp.float32),
               jax.ShapeDtypeStruct((N, DA), jnp.float32)),
)


_mesh = plsc.VectorSubcoreMesh(core_axis_name="c", subcore_axis_name="s")


@functools.partial(
    pl.kernel,
    out_type=jax.ShapeDtypeStruct((NC, ACC_ROWS, DA), jnp.float32),
    mesh=_mesh,
    scratch_types=[
        pltpu.VMEM((2, 2, G), jnp.int32),
        pltpu.VMEM((2, G, DA), jnp.float32),
        pltpu.VMEM_SHARED((ACC_ROWS, DA), jnp.float32),
        pltpu.SemaphoreType.DMA,
        pltpu.SemaphoreType.DMA,
        pltpu.SemaphoreType.DMA,
        pltpu.SemaphoreType.DMA,
    ],
    compiler_params=pltpu.CompilerParams(use_tc_tiling_on_sc=False),
)
def _sc_scatter(xa_hbm, idx_hbm, zeros_hbm, out_hbm,
                idx_v, rows_v, acc_sh, isem0, isem1, rsem0, rsem1):
    c = lax.axis_index("c")
    s = lax.axis_index("s")
    wid = s * NC + c
    base = wid * GPT
    isems = (isem0, isem1)
    rsems = (rsem0, rsem1)

    pltpu.async_copy(idx_hbm.at[base], idx_v.at[0], isem0)
    pltpu.async_copy(idx_hbm.at[base + 1], idx_v.at[1], isem1)
    pltpu.sync_copy(zeros_hbm.at[pl.ds(s * ROWS_PT, ROWS_PT)],
                    acc_sh.at[pl.ds(s * ROWS_PT, ROWS_PT)])
    plsc.subcore_barrier()

    pltpu.make_async_copy(idx_hbm.at[base], idx_v.at[0], isem0).wait()
    pltpu.async_copy(xa_hbm.at[idx_v.at[0].at[0]], rows_v.at[0], rsem0)

    def step(g, p):
        q = 1 - p
        pltpu.make_async_copy(xa_hbm.at[idx_v.at[p].at[0]], rows_v.at[p],
                              rsems[p]).wait()

        @pl.when(g + 1 < GPT)
        def _():
            pltpu.make_async_copy(idx_hbm.at[base + g + 1], idx_v.at[q],
                                  isems[q]).wait()
            pltpu.async_copy(xa_hbm.at[idx_v.at[q].at[0]], rows_v.at[q],
                             rsems[q])

        pltpu.sync_copy(rows_v.at[p], acc_sh.at[idx_v.at[p].at[1]], add=True)

        @pl.when(g + 2 < GPT)
        def _():
            pltpu.async_copy(idx_hbm.at[base + g + 2], idx_v.at[p], isems[p])

    def body(gg, carry):
        g = gg * 2
        step(g, 0)

        @pl.when(g + 1 < GPT)
        def _():
            step(g + 1, 1)

        return carry

    lax.fori_loop(0, (GPT + 1) // 2, body, 0)
    plsc.subcore_barrier()
    pltpu.sync_copy(acc_sh.at[pl.ds(s * ROWS_PT, ROWS_PT)],
                    out_hbm.at[c].at[pl.ds(s * ROWS_PT, ROWS_PT)])


_BN = 2000

_DN = (((1,), (1,)), ((), ()))


def _head_body(a0_ref, a1_ref, xn_ref, wl_ref, bl_ref, wr_ref,
               wc1_ref, bc1_ref, wc2_ref, bc2_ref, out_ref):
    a = a0_ref[...] + a1_ref[...]
    ssum = a[:, :D]
    cnt = a[:, D:D + 1]
    agg = ssum / jnp.maximum(cnt, 1.0)
    xn = xn_ref[...]
    hp = lax.Precision.HIGHEST
    h = (lax.dot_general(agg, wl_ref[...], _DN, precision=hp)
         + lax.dot_general(xn, wr_ref[...], _DN, precision=hp)
         + bl_ref[...])
    h = jnp.maximum(h, 0.0)
    h1 = jnp.maximum(
        lax.dot_general(h, wc1_ref[...], _DN, precision=hp) + bc1_ref[...],
        0.0)
    out_ref[...] = (lax.dot_general(h1, wc2_ref[...], _DN, precision=hp)
                    + bc2_ref[...])


_head_call = pl.pallas_call(
    _head_body,
    grid=(N // _BN,),
    in_specs=[
        pl.BlockSpec((_BN, DA), lambda i: (i, 0)),
        pl.BlockSpec((_BN, DA), lambda i: (i, 0)),
        pl.BlockSpec((_BN, D), lambda i: (i, 0)),
        pl.BlockSpec((H, D), lambda i: (0, 0)),
        pl.BlockSpec((1, H), lambda i: (0, 0)),
        pl.BlockSpec((H, D), lambda i: (0, 0)),
        pl.BlockSpec((16, H), lambda i: (0, 0)),
        pl.BlockSpec((1, 16), lambda i: (0, 0)),
        pl.BlockSpec((2, 16), lambda i: (0, 0)),
        pl.BlockSpec((1, 2), lambda i: (0, 0)),
    ],
    out_specs=pl.BlockSpec((_BN, 2), lambda i: (i, 0)),
    out_shape=jax.ShapeDtypeStruct((N, 2), jnp.float32),
)


def kernel(x, edge_index, edge_weight, edge_features, adj, T,
           bn_weight, bn_bias, W_l, b_l, W_r, Wc1, bc1, Wc2, bc2):
    xn, xa = _bn_call(x, bn_weight.reshape(1, D), bn_bias.reshape(1, D))

    pad = E_PAD - E
    src = jnp.concatenate([edge_index[0], jnp.zeros((pad,), jnp.int32)])
    dump = N + jnp.arange(pad, dtype=jnp.int32) % (ACC_ROWS - N)
    dst = jnp.concatenate([edge_index[1], dump])
    idx3 = jnp.concatenate([src.reshape(NW * GPT, 1, G),
                            dst.reshape(NW * GPT, 1, G)], axis=1)
    zeros = jnp.zeros((ACC_ROWS, DA), jnp.float32)

    acc = _sc_scatter(xa, idx3, zeros)

    out = _head_call(acc[0], acc[1], xn, W_l, b_l.reshape(1, H), W_r,
                     Wc1, bc1.reshape(1, 16), Wc2, bc2.reshape(1, 2))
    return out


# speedup vs baseline: 9.1377x; 1.6998x over previous
import functools

import jax
import jax.numpy as jnp
from jax import lax
from jax.experimental import pallas as pl
from jax.experimental.pallas import tpu as pltpu
from jax.experimental.pallas import tpu_sc as plsc

N = 10000
E = 320000
D = 128
DA = 144
H = 128
G = 128

_info = plsc.get_sparse_core_info()
NC = _info.num_cores
NS = _info.num_subcores
NW = NC * NS
GPT = -(-E // (G * NW))
E_PAD = GPT * G * NW
ROWS_PT = 632
ACC_ROWS = ROWS_PT * NS


def _bn_body(x_ref, w_ref, b_ref, xn_ref, xa_ref):
    x = x_ref[...]
    mu = jnp.mean(x, axis=0, keepdims=True)
    xc = x - mu
    var = jnp.mean(xc * xc, axis=0, keepdims=True)
    xn = xc * lax.rsqrt(var + 1e-5) * w_ref[...] + b_ref[...]
    xn_ref[...] = xn
    xa_ref[...] = jnp.concatenate(
        [xn, jnp.ones((x.shape[0], DA - D), jnp.float32)], axis=1)


_bn_call = pl.pallas_call(
    _bn_body,
    out_shape=(jax.ShapeDtypeStruct((N, D), jnp.float32),
               jax.ShapeDtypeStruct((N, DA), jnp.float32)),
)


_mesh = plsc.VectorSubcoreMesh(core_axis_name="c", subcore_axis_name="s")


@functools.partial(
    pl.kernel,
    out_type=jax.ShapeDtypeStruct((NC, ACC_ROWS, DA), jnp.float32),
    mesh=_mesh,
    scratch_types=[
        pltpu.VMEM((2, G), jnp.int32),
        pltpu.VMEM((2, G), jnp.int32),
        pltpu.VMEM((2, G, DA), jnp.float32),
        pltpu.VMEM_SHARED((ACC_ROWS, DA), jnp.float32),
        pltpu.SemaphoreType.DMA,
        pltpu.SemaphoreType.DMA,
        pltpu.SemaphoreType.DMA,
        pltpu.SemaphoreType.DMA,
    ],
    compiler_params=pltpu.CompilerParams(use_tc_tiling_on_sc=False),
)
def _sc_scatter(xa_hbm, src_hbm, dst_hbm, zeros_hbm, out_hbm,
                src_v, dst_v, rows_v, acc_sh, isem0, isem1, rsem0, rsem1):
    c = lax.axis_index("c")
    s = lax.axis_index("s")
    wid = s * NC + c
    base = wid * GPT
    isems = (isem0, isem1)
    rsems = (rsem0, rsem1)

    def idx_fetch(g, p):
        pltpu.async_copy(src_hbm.at[base + g], src_v.at[p], isems[p])
        pltpu.async_copy(dst_hbm.at[base + g], dst_v.at[p], isems[p])

    def idx_wait(g, p):
        pltpu.make_async_copy(src_hbm.at[base + g], src_v.at[p],
                              isems[p]).wait()
        pltpu.make_async_copy(dst_hbm.at[base + g], dst_v.at[p],
                              isems[p]).wait()

    idx_fetch(0, 0)
    idx_fetch(1, 1)
    pltpu.sync_copy(zeros_hbm.at[pl.ds(s * ROWS_PT, ROWS_PT)],
                    acc_sh.at[pl.ds(s * ROWS_PT, ROWS_PT)])
    plsc.subcore_barrier()

    idx_wait(0, 0)
    pltpu.async_copy(xa_hbm.at[src_v.at[0]], rows_v.at[0], rsem0)

    def step(g, p):
        q = 1 - p
        pltpu.make_async_copy(xa_hbm.at[src_v.at[p]], rows_v.at[p],
                              rsems[p]).wait()

        @pl.when(g + 1 < GPT)
        def _():
            idx_wait(g + 1, q)
            pltpu.async_copy(xa_hbm.at[src_v.at[q]], rows_v.at[q],
                             rsems[q])

        pltpu.sync_copy(rows_v.at[p], acc_sh.at[dst_v.at[p]], add=True)

        @pl.when(g + 2 < GPT)
        def _():
            idx_fetch(g + 2, p)

    def body(gg, carry):
        g = gg * 2
        step(g, 0)

        @pl.when(g + 1 < GPT)
        def _():
            step(g + 1, 1)

        return carry

    lax.fori_loop(0, (GPT + 1) // 2, body, 0)
    plsc.subcore_barrier()
    pltpu.sync_copy(acc_sh.at[pl.ds(s * ROWS_PT, ROWS_PT)],
                    out_hbm.at[c].at[pl.ds(s * ROWS_PT, ROWS_PT)])


_BN = 2000

_DN = (((1,), (1,)), ((), ()))


def _head_body(a0_ref, a1_ref, xn_ref, wl_ref, bl_ref, wr_ref,
               wc1_ref, bc1_ref, wc2_ref, bc2_ref, out_ref):
    a = a0_ref[0] + a1_ref[0]
    ssum = a[:, :D]
    cnt = a[:, D:D + 1]
    agg = ssum / jnp.maximum(cnt, 1.0)
    xn = xn_ref[...]
    hp = lax.Precision.HIGHEST
    h = (lax.dot_general(agg, wl_ref[...], _DN, precision=hp)
         + lax.dot_general(xn, wr_ref[...], _DN, precision=hp)
         + bl_ref[...])
    h = jnp.maximum(h, 0.0)
    h1 = jnp.maximum(
        lax.dot_general(h, wc1_ref[...], _DN, precision=hp) + bc1_ref[...],
        0.0)
    out_ref[...] = (lax.dot_general(h1, wc2_ref[...], _DN, precision=hp)
                    + bc2_ref[...])


_head_call = pl.pallas_call(
    _head_body,
    grid=(N // _BN,),
    in_specs=[
        pl.BlockSpec((1, _BN, DA), lambda i: (0, i, 0)),
        pl.BlockSpec((1, _BN, DA), lambda i: (1, i, 0)),
        pl.BlockSpec((_BN, D), lambda i: (i, 0)),
        pl.BlockSpec((H, D), lambda i: (0, 0)),
        pl.BlockSpec((1, H), lambda i: (0, 0)),
        pl.BlockSpec((H, D), lambda i: (0, 0)),
        pl.BlockSpec((16, H), lambda i: (0, 0)),
        pl.BlockSpec((1, 16), lambda i: (0, 0)),
        pl.BlockSpec((2, 16), lambda i: (0, 0)),
        pl.BlockSpec((1, 2), lambda i: (0, 0)),
    ],
    out_specs=pl.BlockSpec((_BN, 2), lambda i: (i, 0)),
    out_shape=jax.ShapeDtypeStruct((N, 2), jnp.float32),
)


def kernel(x, edge_index, edge_weight, edge_features, adj, T,
           bn_weight, bn_bias, W_l, b_l, W_r, Wc1, bc1, Wc2, bc2):
    xn, xa = _bn_call(x, bn_weight.reshape(1, D), bn_bias.reshape(1, D))

    pad = E_PAD - E
    fill = jnp.arange(pad, dtype=jnp.int32)
    src = jnp.concatenate([edge_index[0], fill % N]).reshape(NW * GPT, G)
    dst = jnp.concatenate([edge_index[1],
                           N + fill % (ACC_ROWS - N)]).reshape(NW * GPT, G)
    zeros = jnp.zeros((ACC_ROWS, DA), jnp.float32)

    acc = _sc_scatter(xa, src, dst, zeros)

    out = _head_call(acc, acc, xn, W_l, b_l.reshape(1, H), W_r,
                     Wc1, bc1.reshape(1, 16), Wc2, bc2.reshape(1, 2))
    return out


# speedup vs baseline: 10.2403x; 1.1207x over previous
import functools

import jax
import jax.numpy as jnp
from jax import lax
from jax.experimental import pallas as pl
from jax.experimental.pallas import tpu as pltpu
from jax.experimental.pallas import tpu_sc as plsc

N = 10000
E = 320000
D = 128
H = 128
G = 128

_info = plsc.get_sparse_core_info()
NC = _info.num_cores
NS = _info.num_subcores
NW = NC * NS
NG = E // G
GPT_LO = NG // NW
N_HI = NG % NW
HR = 79
ROWS_PT = 632
ACC_ROWS = ROWS_PT * NS


def _bn_body(x_ref, w_ref, b_ref, xn_ref):
    x = x_ref[...]
    mu = jnp.mean(x, axis=0, keepdims=True)
    xc = x - mu
    var = jnp.mean(xc * xc, axis=0, keepdims=True)
    xn_ref[...] = xc * lax.rsqrt(var + 1e-5) * w_ref[...] + b_ref[...]


_bn_call = pl.pallas_call(
    _bn_body,
    out_shape=jax.ShapeDtypeStruct((N, D), jnp.float32),
)


_mesh = plsc.VectorSubcoreMesh(core_axis_name="c", subcore_axis_name="s")


CW = 16


@functools.partial(
    pl.kernel,
    out_type=(jax.ShapeDtypeStruct((NC, ACC_ROWS, D), jnp.float32),
              jax.ShapeDtypeStruct((NC, ACC_ROWS, CW), jnp.float32)),
    mesh=_mesh,
    scratch_types=[
        pltpu.VMEM((2, G), jnp.int32),
        pltpu.VMEM((2, G), jnp.int32),
        pltpu.VMEM((2, G, D), jnp.float32),
        pltpu.VMEM((G, CW), jnp.float32),
        pltpu.VMEM_SHARED((ACC_ROWS, D), jnp.float32),
        pltpu.VMEM_SHARED((ACC_ROWS, CW), jnp.float32),
        pltpu.SemaphoreType.DMA,
        pltpu.SemaphoreType.DMA,
        pltpu.SemaphoreType.DMA,
        pltpu.SemaphoreType.DMA,
    ],
    compiler_params=pltpu.CompilerParams(use_tc_tiling_on_sc=False),
)
def _sc_scatter(xn_hbm, edge_hbm, zeros_hbm, zeros16_hbm, ones_hbm,
                out_hbm, cnt_hbm,
                src_v, dst_v, rows_v, ones_v, acc_sh, cnt_sh,
                isem0, isem1, rsem0, rsem1):
    c = lax.axis_index("c")
    s = lax.axis_index("s")
    wid = s * NC + c
    n_g = GPT_LO + jnp.where(wid < N_HI, 1, 0)
    base = wid * GPT_LO + jnp.minimum(wid, N_HI)
    isems = (isem0, isem1)
    rsems = (rsem0, rsem1)

    def idx_fetch(g, p):
        off = (base + g) * G
        pltpu.async_copy(edge_hbm.at[0].at[pl.ds(off, G)], src_v.at[p],
                         isems[p])
        pltpu.async_copy(edge_hbm.at[1].at[pl.ds(off, G)], dst_v.at[p],
                         isems[p])

    def idx_wait(g, p):
        off = (base + g) * G
        pltpu.make_async_copy(edge_hbm.at[0].at[pl.ds(off, G)], src_v.at[p],
                              isems[p]).wait()
        pltpu.make_async_copy(edge_hbm.at[1].at[pl.ds(off, G)], dst_v.at[p],
                              isems[p]).wait()

    idx_fetch(0, 0)
    idx_fetch(1, 1)
    pltpu.sync_copy(ones_hbm, ones_v)
    pltpu.sync_copy(zeros_hbm.at[pl.ds(s * ROWS_PT, ROWS_PT)],
                    acc_sh.at[pl.ds(s * ROWS_PT, ROWS_PT)])
    pltpu.sync_copy(zeros16_hbm.at[pl.ds(s * ROWS_PT, ROWS_PT)],
                    cnt_sh.at[pl.ds(s * ROWS_PT, ROWS_PT)])
    plsc.subcore_barrier()

    idx_wait(0, 0)
    pltpu.async_copy(xn_hbm.at[src_v.at[0]], rows_v.at[0], rsem0)

    def step(g, p):
        q = 1 - p
        pltpu.make_async_copy(xn_hbm.at[src_v.at[p]], rows_v.at[p],
                              rsems[p]).wait()

        @pl.when(g + 1 < n_g)
        def _():
            idx_wait(g + 1, q)
            pltpu.async_copy(xn_hbm.at[src_v.at[q]], rows_v.at[q],
                             rsems[q])

        pltpu.sync_copy(rows_v.at[p], acc_sh.at[dst_v.at[p]], add=True)
        pltpu.sync_copy(ones_v, cnt_sh.at[dst_v.at[p]], add=True)

        @pl.when(g + 2 < n_g)
        def _():
            idx_fetch(g + 2, p)

    def body(gg, carry):
        g = gg * 2
        step(g, 0)

        @pl.when(g + 1 < n_g)
        def _():
            step(g + 1, 1)

        return carry

    lax.fori_loop(0, (n_g + 1) // 2, body, 0)
    plsc.subcore_barrier()
    pltpu.sync_copy(acc_sh.at[pl.ds(s * ROWS_PT, ROWS_PT)],
                    out_hbm.at[c].at[pl.ds(s * ROWS_PT, ROWS_PT)])
    pltpu.sync_copy(cnt_sh.at[pl.ds(s * ROWS_PT, ROWS_PT)],
                    cnt_hbm.at[c].at[pl.ds(s * ROWS_PT, ROWS_PT)])


_DN = (((1,), (1,)), ((), ()))


def _head_body(a_ref, cnt_ref, xn_ref, wl_ref, bl_ref, wr_ref,
               wc1_ref, bc1_ref, wc2_ref, bc2_ref, out_ref):
    a = a_ref[0] + a_ref[1]
    cnt = cnt_ref[...]
    agg = a[:N] / jnp.maximum(cnt, 1.0)
    xn = xn_ref[...]
    hp = lax.Precision.HIGHEST
    h = (lax.dot_general(agg, wl_ref[...], _DN, precision=hp)
         + lax.dot_general(xn, wr_ref[...], _DN, precision=hp)
         + bl_ref[...])
    h = jnp.maximum(h, 0.0)
    h1 = jnp.maximum(
        lax.dot_general(h, wc1_ref[...], _DN, precision=hp) + bc1_ref[...],
        0.0)
    out_ref[...] = (lax.dot_general(h1, wc2_ref[...], _DN, precision=hp)
                    + bc2_ref[...])


_head_call = pl.pallas_call(
    _head_body,
    out_shape=jax.ShapeDtypeStruct((N, 2), jnp.float32),
)


def kernel(x, edge_index, edge_weight, edge_features, adj, T,
           bn_weight, bn_bias, W_l, b_l, W_r, Wc1, bc1, Wc2, bc2):
    xn = _bn_call(x, bn_weight.reshape(1, D), bn_bias.reshape(1, D))
    zeros = jnp.zeros((ACC_ROWS, D), jnp.float32)
    zeros16 = jnp.zeros((ACC_ROWS, CW), jnp.float32)
    ones = jnp.ones((G, CW), jnp.float32)
    acc, cnts = _sc_scatter(xn, edge_index, zeros, zeros16, ones)
    cnt = (cnts[0, :N, :1] + cnts[1, :N, :1])
    out = _head_call(acc, cnt, xn, W_l, b_l.reshape(1, H), W_r,
                     Wc1, bc1.reshape(1, 16), Wc2, bc2.reshape(1, 2))
    return out


# speedup vs baseline: 10.3543x; 1.0111x over previous
import functools

import jax
import jax.numpy as jnp
from jax import lax
from jax.experimental import pallas as pl
from jax.experimental.pallas import tpu as pltpu
from jax.experimental.pallas import tpu_sc as plsc

N = 10000
E = 320000
D = 128
H = 128
G = 128

_info = plsc.get_sparse_core_info()
NC = _info.num_cores
NS = _info.num_subcores
NW = NC * NS
NG = E // G
GPT_LO = NG // NW
N_HI = NG % NW
HR = 79
ROWS_PT = 632
ACC_ROWS = ROWS_PT * NS


def _bn_body(x_ref, w_ref, b_ref, xn_ref):
    x = x_ref[...]
    mu = jnp.mean(x, axis=0, keepdims=True)
    xc = x - mu
    var = jnp.mean(xc * xc, axis=0, keepdims=True)
    xn_ref[...] = xc * lax.rsqrt(var + 1e-5) * w_ref[...] + b_ref[...]


_bn_call = pl.pallas_call(
    _bn_body,
    out_shape=jax.ShapeDtypeStruct((N, D), jnp.float32),
)


_mesh = plsc.VectorSubcoreMesh(core_axis_name="c", subcore_axis_name="s")


CW = 16


@functools.partial(
    pl.kernel,
    out_type=(jax.ShapeDtypeStruct((NC, ACC_ROWS, D), jnp.float32),
              jax.ShapeDtypeStruct((NC, ACC_ROWS, CW), jnp.float32)),
    mesh=_mesh,
    scratch_types=[
        pltpu.VMEM((4, G), jnp.int32),
        pltpu.VMEM((4, G), jnp.int32),
        pltpu.VMEM((2, G, D), jnp.float32),
        pltpu.VMEM((G, CW), jnp.float32),
        pltpu.VMEM_SHARED((ACC_ROWS, D), jnp.float32),
        pltpu.VMEM_SHARED((ACC_ROWS, CW), jnp.float32),
        pltpu.SemaphoreType.DMA,
        pltpu.SemaphoreType.DMA,
        pltpu.SemaphoreType.DMA,
        pltpu.SemaphoreType.DMA,
        pltpu.SemaphoreType.DMA,
        pltpu.SemaphoreType.DMA,
        pltpu.SemaphoreType.DMA,
        pltpu.SemaphoreType.DMA,
        pltpu.SemaphoreType.DMA,
        pltpu.SemaphoreType.DMA,
    ],
    compiler_params=pltpu.CompilerParams(use_tc_tiling_on_sc=False),
)
def _sc_scatter(xn_hbm, edge_hbm, zeros_hbm, zeros16_hbm, ones_hbm,
                out_hbm, cnt_hbm,
                src_v, dst_v, rows_v, ones_v, acc_sh, cnt_sh,
                isem0, isem1, isem2, isem3,
                grsem0, grsem1, fsem0, fsem1, csem0, csem1):
    c = lax.axis_index("c")
    s = lax.axis_index("s")
    wid = s * NC + c
    n_g = GPT_LO + jnp.where(wid < N_HI, 1, 0)
    base = wid * GPT_LO + jnp.minimum(wid, N_HI)
    isems = (isem0, isem1, isem2, isem3)
    grsems = (grsem0, grsem1)
    fsems = (fsem0, fsem1)
    csems = (csem0, csem1)

    def idx_fetch(g, r):
        off = (base + g) * G
        pltpu.async_copy(edge_hbm.at[0].at[pl.ds(off, G)], src_v.at[r],
                         isems[r])
        pltpu.async_copy(edge_hbm.at[1].at[pl.ds(off, G)], dst_v.at[r],
                         isems[r])

    def idx_wait(g, r):
        off = (base + g) * G
        pltpu.make_async_copy(edge_hbm.at[0].at[pl.ds(off, G)], src_v.at[r],
                              isems[r]).wait()
        pltpu.make_async_copy(edge_hbm.at[1].at[pl.ds(off, G)], dst_v.at[r],
                              isems[r]).wait()

    def feat_wait(p):
        pltpu.make_async_copy(rows_v.at[p], acc_sh.at[dst_v.at[0]],
                              fsems[p]).wait()

    def cnt_wait(p):
        pltpu.make_async_copy(ones_v, cnt_sh.at[dst_v.at[0]],
                              csems[p]).wait()

    idx_fetch(0, 0)
    idx_fetch(1, 1)
    idx_fetch(2, 2)
    idx_fetch(3, 3)
    pltpu.sync_copy(ones_hbm, ones_v)
    pltpu.sync_copy(zeros_hbm.at[pl.ds(s * ROWS_PT, ROWS_PT)],
                    acc_sh.at[pl.ds(s * ROWS_PT, ROWS_PT)])
    pltpu.sync_copy(zeros16_hbm.at[pl.ds(s * ROWS_PT, ROWS_PT)],
                    cnt_sh.at[pl.ds(s * ROWS_PT, ROWS_PT)])
    plsc.subcore_barrier()

    idx_wait(0, 0)
    pltpu.async_copy(xn_hbm.at[src_v.at[0]], rows_v.at[0], grsem0)

    def step(g, k):
        p = k % 2
        q = 1 - p
        pltpu.make_async_copy(xn_hbm.at[src_v.at[k]], rows_v.at[p],
                              grsems[p]).wait()

        @pl.when(g >= 1)
        def _():
            feat_wait(q)
            cnt_wait(q)

        @pl.when(g + 1 < n_g)
        def _():
            idx_wait(g + 1, (k + 1) % 4)
            pltpu.async_copy(xn_hbm.at[src_v.at[(k + 1) % 4]], rows_v.at[q],
                             grsems[q])

        pltpu.async_copy(rows_v.at[p], acc_sh.at[dst_v.at[k]], fsems[p],
                         add=True)
        pltpu.async_copy(ones_v, cnt_sh.at[dst_v.at[k]], csems[p],
                         add=True)

        @pl.when((g + 3 < n_g) & (g >= 1))
        def _():
            idx_fetch(g + 3, (k + 3) % 4)

    def body(ii, carry):
        g0 = ii * 4
        step(g0, 0)
        for k in (1, 2, 3):
            @pl.when(g0 + k < n_g)
            def _(k=k):
                step(g0 + k, k)

        return carry

    lax.fori_loop(0, (n_g + 3) // 4, body, 0)
    last_p = (n_g - 1) % 2

    @pl.when(last_p == 0)
    def _():
        feat_wait(0)
        cnt_wait(0)

    @pl.when(last_p == 1)
    def _():
        feat_wait(1)
        cnt_wait(1)

    plsc.subcore_barrier()
    pltpu.sync_copy(acc_sh.at[pl.ds(s * ROWS_PT, ROWS_PT)],
                    out_hbm.at[c].at[pl.ds(s * ROWS_PT, ROWS_PT)])
    pltpu.sync_copy(cnt_sh.at[pl.ds(s * ROWS_PT, ROWS_PT)],
                    cnt_hbm.at[c].at[pl.ds(s * ROWS_PT, ROWS_PT)])


_DN = (((1,), (1,)), ((), ()))


def _head_body(a_ref, cnt_ref, xn_ref, wl_ref, bl_ref, wr_ref,
               wc1_ref, bc1_ref, wc2_ref, bc2_ref, out_ref):
    a = a_ref[0] + a_ref[1]
    cnt = cnt_ref[...]
    agg = a[:N] / jnp.maximum(cnt, 1.0)
    xn = xn_ref[...]
    hp = lax.Precision.HIGHEST
    h = (lax.dot_general(agg, wl_ref[...], _DN, precision=hp)
         + lax.dot_general(xn, wr_ref[...], _DN, precision=hp)
         + bl_ref[...])
    h = jnp.maximum(h, 0.0)
    h1 = jnp.maximum(
        lax.dot_general(h, wc1_ref[...], _DN, precision=hp) + bc1_ref[...],
        0.0)
    out_ref[...] = (lax.dot_general(h1, wc2_ref[...], _DN, precision=hp)
                    + bc2_ref[...])


_head_call = pl.pallas_call(
    _head_body,
    out_shape=jax.ShapeDtypeStruct((N, 2), jnp.float32),
)


def kernel(x, edge_index, edge_weight, edge_features, adj, T,
           bn_weight, bn_bias, W_l, b_l, W_r, Wc1, bc1, Wc2, bc2):
    xn = _bn_call(x, bn_weight.reshape(1, D), bn_bias.reshape(1, D))
    zeros = jnp.zeros((ACC_ROWS, D), jnp.float32)
    zeros16 = jnp.zeros((ACC_ROWS, CW), jnp.float32)
    ones = jnp.ones((G, CW), jnp.float32)
    acc, cnts = _sc_scatter(xn, edge_index, zeros, zeros16, ones)
    cnt = (cnts[0, :N, :1] + cnts[1, :N, :1])
    out = _head_call(acc, cnt, xn, W_l, b_l.reshape(1, H), W_r,
                     Wc1, bc1.reshape(1, 16), Wc2, bc2.reshape(1, 2))
    return out


# speedup vs baseline: 11.8155x; 1.1411x over previous
import functools

import jax
import jax.numpy as jnp
from jax import lax
from jax.experimental import pallas as pl
from jax.experimental.pallas import tpu as pltpu
from jax.experimental.pallas import tpu_sc as plsc

N = 10000
E = 320000
D = 128
H = 128
G = 128

_info = plsc.get_sparse_core_info()
NC = _info.num_cores
NS = _info.num_subcores
NW = NC * NS
NG = E // G
GPT_LO = NG // NW
N_HI = NG % NW
HR = 79
ROWS_PT = 632
ACC_ROWS = ROWS_PT * NS


def _bn_body(x_ref, w_ref, b_ref, xn_ref):
    x = x_ref[...]
    mu = jnp.mean(x, axis=0, keepdims=True)
    xc = x - mu
    var = jnp.mean(xc * xc, axis=0, keepdims=True)
    xn_ref[...] = xc * lax.rsqrt(var + 1e-5) * w_ref[...] + b_ref[...]


_bn_call = pl.pallas_call(
    _bn_body,
    out_shape=jax.ShapeDtypeStruct((N, D), jnp.float32),
)


_mesh = plsc.VectorSubcoreMesh(core_axis_name="c", subcore_axis_name="s")


CW = 16


@functools.partial(
    pl.kernel,
    out_type=(jax.ShapeDtypeStruct((NC, ACC_ROWS, D), jnp.float32),
              jax.ShapeDtypeStruct((NC, ACC_ROWS, CW), jnp.float32)),
    mesh=_mesh,
    scratch_types=[
        pltpu.VMEM((4, G), jnp.int32),
        pltpu.VMEM((4, G), jnp.int32),
        pltpu.VMEM((2, G, D), jnp.float32),
        pltpu.VMEM((G, CW), jnp.float32),
        pltpu.VMEM_SHARED((ACC_ROWS, D), jnp.float32),
        pltpu.VMEM_SHARED((ACC_ROWS, CW), jnp.float32),
        pltpu.SemaphoreType.DMA,
        pltpu.SemaphoreType.DMA,
        pltpu.SemaphoreType.DMA,
        pltpu.SemaphoreType.DMA,
        pltpu.SemaphoreType.DMA,
        pltpu.SemaphoreType.DMA,
        pltpu.SemaphoreType.DMA,
        pltpu.SemaphoreType.DMA,
        pltpu.SemaphoreType.DMA,
        pltpu.SemaphoreType.DMA,
    ],
    compiler_params=pltpu.CompilerParams(use_tc_tiling_on_sc=False),
)
def _sc_scatter(xn_hbm, edge_hbm, zeros_hbm, zeros16_hbm, ones_hbm,
                out_hbm, cnt_hbm,
                src_v, dst_v, rows_v, ones_v, acc_sh, cnt_sh,
                isem0, isem1, isem2, isem3,
                grsem0, grsem1, fsem0, fsem1, csem0, csem1):
    c = lax.axis_index("c")
    s = lax.axis_index("s")
    wid = s * NC + c
    n_g = GPT_LO + jnp.where(wid < N_HI, 1, 0)
    base = wid * GPT_LO + jnp.minimum(wid, N_HI)
    isems = (isem0, isem1, isem2, isem3)
    grsems = (grsem0, grsem1)
    fsems = (fsem0, fsem1)
    csems = (csem0, csem1)

    def idx_fetch(g, r):
        off = (base + g) * G
        pltpu.async_copy(edge_hbm.at[0].at[pl.ds(off, G)], src_v.at[r],
                         isems[r])
        pltpu.async_copy(edge_hbm.at[1].at[pl.ds(off, G)], dst_v.at[r],
                         isems[r])

    def idx_wait(g, r):
        off = (base + g) * G
        pltpu.make_async_copy(edge_hbm.at[0].at[pl.ds(off, G)], src_v.at[r],
                              isems[r]).wait()
        pltpu.make_async_copy(edge_hbm.at[1].at[pl.ds(off, G)], dst_v.at[r],
                              isems[r]).wait()

    def feat_wait(p):
        pltpu.make_async_copy(rows_v.at[p], acc_sh.at[dst_v.at[0]],
                              fsems[p]).wait()

    def cnt_wait(p):
        pltpu.make_async_copy(ones_v, cnt_sh.at[dst_v.at[0]],
                              csems[p]).wait()

    idx_fetch(0, 0)
    idx_fetch(1, 1)
    idx_fetch(2, 2)
    idx_fetch(3, 3)
    pltpu.sync_copy(ones_hbm, ones_v)
    pltpu.sync_copy(zeros_hbm.at[pl.ds(s * ROWS_PT, ROWS_PT)],
                    acc_sh.at[pl.ds(s * ROWS_PT, ROWS_PT)])
    pltpu.sync_copy(zeros16_hbm.at[pl.ds(s * ROWS_PT, ROWS_PT)],
                    cnt_sh.at[pl.ds(s * ROWS_PT, ROWS_PT)])
    plsc.subcore_barrier()

    idx_wait(0, 0)
    pltpu.async_copy(xn_hbm.at[src_v.at[0]], rows_v.at[0], grsem0)

    def step(g, k):
        p = k % 2
        q = 1 - p
        pltpu.make_async_copy(xn_hbm.at[src_v.at[k]], rows_v.at[p],
                              grsems[p]).wait()

        @pl.when(g >= 1)
        def _():
            feat_wait(q)
            cnt_wait(q)

        @pl.when(g + 1 < n_g)
        def _():
            idx_wait(g + 1, (k + 1) % 4)
            pltpu.async_copy(xn_hbm.at[src_v.at[(k + 1) % 4]], rows_v.at[q],
                             grsems[q])

        pltpu.async_copy(rows_v.at[p], acc_sh.at[dst_v.at[k]], fsems[p],
                         add=True)
        pltpu.async_copy(ones_v, cnt_sh.at[dst_v.at[k]], csems[p],
                         add=True)

        @pl.when((g + 3 < n_g) & (g >= 1))
        def _():
            idx_fetch(g + 3, (k + 3) % 4)

    def body(ii, carry):
        g0 = ii * 4
        step(g0, 0)
        for k in (1, 2, 3):
            @pl.when(g0 + k < n_g)
            def _(k=k):
                step(g0 + k, k)

        return carry

    lax.fori_loop(0, (n_g + 3) // 4, body, 0)
    last_p = (n_g - 1) % 2

    @pl.when(last_p == 0)
    def _():
        feat_wait(0)
        cnt_wait(0)

    @pl.when(last_p == 1)
    def _():
        feat_wait(1)
        cnt_wait(1)

    plsc.subcore_barrier()
    pltpu.sync_copy(acc_sh.at[pl.ds(s * ROWS_PT, ROWS_PT)],
                    out_hbm.at[c].at[pl.ds(s * ROWS_PT, ROWS_PT)])
    pltpu.sync_copy(cnt_sh.at[pl.ds(s * ROWS_PT, ROWS_PT)],
                    cnt_hbm.at[c].at[pl.ds(s * ROWS_PT, ROWS_PT)])


_DN = (((1,), (1,)), ((), ()))


def _head_body(a_ref, cnt_ref, xn_ref, wl_ref, bl_ref, wr_ref,
               wc1_ref, bc1_ref, wc2_ref, bc2_ref, out_ref):
    a = a_ref[0] + a_ref[1]
    cnt = cnt_ref[...]
    agg = a / jnp.maximum(cnt, 1.0)
    xn = xn_ref[...]
    hp = lax.Precision.DEFAULT
    h = (lax.dot_general(agg, wl_ref[...], _DN, precision=hp)
         + lax.dot_general(xn, wr_ref[...], _DN, precision=hp)
         + bl_ref[...])
    h = jnp.maximum(h, 0.0)
    h1 = jnp.maximum(
        lax.dot_general(h, wc1_ref[...], _DN, precision=hp) + bc1_ref[...],
        0.0)
    out_ref[...] = (lax.dot_general(h1, wc2_ref[...], _DN, precision=hp)
                    + bc2_ref[...])


_BN = 2000

_head_call = pl.pallas_call(
    _head_body,
    grid=(N // _BN,),
    in_specs=[
        pl.BlockSpec((2, _BN, D), lambda i: (0, i, 0)),
        pl.BlockSpec((_BN, 1), lambda i: (i, 0)),
        pl.BlockSpec((_BN, D), lambda i: (i, 0)),
        pl.BlockSpec((H, D), lambda i: (0, 0)),
        pl.BlockSpec((1, H), lambda i: (0, 0)),
        pl.BlockSpec((H, D), lambda i: (0, 0)),
        pl.BlockSpec((16, H), lambda i: (0, 0)),
        pl.BlockSpec((1, 16), lambda i: (0, 0)),
        pl.BlockSpec((2, 16), lambda i: (0, 0)),
        pl.BlockSpec((1, 2), lambda i: (0, 0)),
    ],
    out_specs=pl.BlockSpec((_BN, 2), lambda i: (i, 0)),
    out_shape=jax.ShapeDtypeStruct((N, 2), jnp.float32),
)


def kernel(x, edge_index, edge_weight, edge_features, adj, T,
           bn_weight, bn_bias, W_l, b_l, W_r, Wc1, bc1, Wc2, bc2):
    xn = _bn_call(x, bn_weight.reshape(1, D), bn_bias.reshape(1, D))
    zeros = jnp.zeros((ACC_ROWS, D), jnp.float32)
    zeros16 = jnp.zeros((ACC_ROWS, CW), jnp.float32)
    ones = jnp.ones((G, CW), jnp.float32)
    acc, cnts = _sc_scatter(xn, edge_index, zeros, zeros16, ones)
    cnt = (cnts[0, :N, :1] + cnts[1, :N, :1])
    out = _head_call(acc, cnt, xn, W_l, b_l.reshape(1, H), W_r,
                     Wc1, bc1.reshape(1, 16), Wc2, bc2.reshape(1, 2))
    return out


# speedup vs baseline: 12.1428x; 1.0277x over previous
import functools

import jax
import jax.numpy as jnp
from jax import lax
from jax.experimental import pallas as pl
from jax.experimental.pallas import tpu as pltpu
from jax.experimental.pallas import tpu_sc as plsc

N = 10000
E = 320000
D = 128
H = 128
G = 128

_info = plsc.get_sparse_core_info()
NC = _info.num_cores
NS = _info.num_subcores
NW = NC * NS
NG = E // G
GPT_LO = NG // NW
N_HI = NG % NW
HR = 79
ROWS_PT = 632
ACC_ROWS = ROWS_PT * NS


def _bn_body(x_ref, w_ref, b_ref, xn_ref):
    x = x_ref[...]
    mu = jnp.mean(x, axis=0, keepdims=True)
    xc = x - mu
    var = jnp.mean(xc * xc, axis=0, keepdims=True)
    xn_ref[...] = xc * lax.rsqrt(var + 1e-5) * w_ref[...] + b_ref[...]


_bn_call = pl.pallas_call(
    _bn_body,
    out_shape=jax.ShapeDtypeStruct((N, D), jnp.float32),
)


_mesh = plsc.VectorSubcoreMesh(core_axis_name="c", subcore_axis_name="s")


CW = 16


@functools.partial(
    pl.kernel,
    out_type=(jax.ShapeDtypeStruct((NC, ACC_ROWS, D), jnp.float32),
              jax.ShapeDtypeStruct((NC, ACC_ROWS, CW), jnp.float32)),
    mesh=_mesh,
    scratch_types=[
        pltpu.VMEM((4, G), jnp.int32),
        pltpu.VMEM((4, G), jnp.int32),
        pltpu.VMEM((2, G, D), jnp.float32),
        pltpu.VMEM((G, CW), jnp.float32),
        pltpu.VMEM_SHARED((ACC_ROWS, D), jnp.float32),
        pltpu.VMEM_SHARED((ACC_ROWS, CW), jnp.float32),
        pltpu.SemaphoreType.DMA,
        pltpu.SemaphoreType.DMA,
        pltpu.SemaphoreType.DMA,
        pltpu.SemaphoreType.DMA,
        pltpu.SemaphoreType.DMA,
        pltpu.SemaphoreType.DMA,
        pltpu.SemaphoreType.DMA,
        pltpu.SemaphoreType.DMA,
        pltpu.SemaphoreType.DMA,
        pltpu.SemaphoreType.DMA,
    ],
    compiler_params=pltpu.CompilerParams(use_tc_tiling_on_sc=False),
)
def _sc_scatter(xn_hbm, edge_hbm, zeros_hbm, zeros16_hbm, ones_hbm,
                out_hbm, cnt_hbm,
                src_v, dst_v, rows_v, ones_v, acc_sh, cnt_sh,
                isem0, isem1, isem2, isem3,
                grsem0, grsem1, fsem0, fsem1, csem0, csem1):
    c = lax.axis_index("c")
    s = lax.axis_index("s")
    wid = s * NC + c
    n_g = GPT_LO + jnp.where(wid < N_HI, 1, 0)
    base = wid * GPT_LO + jnp.minimum(wid, N_HI)
    isems = (isem0, isem1, isem2, isem3)
    grsems = (grsem0, grsem1)
    fsems = (fsem0, fsem1)
    csems = (csem0, csem1)

    def idx_fetch(g, r):
        off = (base + g) * G
        pltpu.async_copy(edge_hbm.at[0].at[pl.ds(off, G)], src_v.at[r],
                         isems[r])
        pltpu.async_copy(edge_hbm.at[1].at[pl.ds(off, G)], dst_v.at[r],
                         isems[r])

    def idx_wait(g, r):
        off = (base + g) * G
        pltpu.make_async_copy(edge_hbm.at[0].at[pl.ds(off, G)], src_v.at[r],
                              isems[r]).wait()
        pltpu.make_async_copy(edge_hbm.at[1].at[pl.ds(off, G)], dst_v.at[r],
                              isems[r]).wait()

    def feat_wait(p):
        pltpu.make_async_copy(rows_v.at[p], acc_sh.at[dst_v.at[0]],
                              fsems[p]).wait()

    def cnt_wait(p):
        pltpu.make_async_copy(ones_v, cnt_sh.at[dst_v.at[0]],
                              csems[p]).wait()

    idx_fetch(0, 0)
    idx_fetch(1, 1)
    idx_fetch(2, 2)
    idx_fetch(3, 3)
    pltpu.sync_copy(ones_hbm, ones_v)
    pltpu.sync_copy(zeros_hbm.at[pl.ds(s * ROWS_PT, ROWS_PT)],
                    acc_sh.at[pl.ds(s * ROWS_PT, ROWS_PT)])
    pltpu.sync_copy(zeros16_hbm.at[pl.ds(s * ROWS_PT, ROWS_PT)],
                    cnt_sh.at[pl.ds(s * ROWS_PT, ROWS_PT)])
    plsc.subcore_barrier()

    idx_wait(0, 0)
    pltpu.async_copy(xn_hbm.at[src_v.at[0]], rows_v.at[0], grsem0)

    def step(g, k):
        p = k % 2
        q = 1 - p
        pltpu.make_async_copy(xn_hbm.at[src_v.at[k]], rows_v.at[p],
                              grsems[p]).wait()

        @pl.when(g >= 1)
        def _():
            feat_wait(q)
            cnt_wait(q)

        @pl.when(g + 1 < n_g)
        def _():
            idx_wait(g + 1, (k + 1) % 4)
            pltpu.async_copy(xn_hbm.at[src_v.at[(k + 1) % 4]], rows_v.at[q],
                             grsems[q])

        pltpu.async_copy(rows_v.at[p], acc_sh.at[dst_v.at[k]], fsems[p],
                         add=True)
        pltpu.async_copy(ones_v, cnt_sh.at[dst_v.at[k]], csems[p],
                         add=True)

        @pl.when((g + 3 < n_g) & (g >= 1))
        def _():
            idx_fetch(g + 3, (k + 3) % 4)

    def body(ii, carry):
        g0 = ii * 4
        step(g0, 0)
        for k in (1, 2, 3):
            @pl.when(g0 + k < n_g)
            def _(k=k):
                step(g0 + k, k)

        return carry

    lax.fori_loop(0, (n_g + 3) // 4, body, 0)
    last_p = (n_g - 1) % 2

    @pl.when(last_p == 0)
    def _():
        feat_wait(0)
        cnt_wait(0)

    @pl.when(last_p == 1)
    def _():
        feat_wait(1)
        cnt_wait(1)

    plsc.subcore_barrier()
    pltpu.sync_copy(acc_sh.at[pl.ds(s * ROWS_PT, ROWS_PT)],
                    out_hbm.at[c].at[pl.ds(s * ROWS_PT, ROWS_PT)])
    pltpu.sync_copy(cnt_sh.at[pl.ds(s * ROWS_PT, ROWS_PT)],
                    cnt_hbm.at[c].at[pl.ds(s * ROWS_PT, ROWS_PT)])


_DN = (((1,), (1,)), ((), ()))


def _head_body(a_ref, cnt_ref, xn_ref, wl_ref, bl_ref, wr_ref,
               wc1_ref, bc1_ref, wc2_ref, bc2_ref, out_ref):
    a = a_ref[0] + a_ref[1]
    cnt = (cnt_ref[0] + cnt_ref[1])[:, :1]
    agg = a / jnp.maximum(cnt, 1.0)
    xn = xn_ref[...]
    hp = lax.Precision.DEFAULT
    h = (lax.dot_general(agg, wl_ref[...], _DN, precision=hp)
         + lax.dot_general(xn, wr_ref[...], _DN, precision=hp)
         + bl_ref[...])
    h = jnp.maximum(h, 0.0)
    h1 = jnp.maximum(
        lax.dot_general(h, wc1_ref[...], _DN, precision=hp) + bc1_ref[...],
        0.0)
    out_ref[...] = (lax.dot_general(h1, wc2_ref[...], _DN, precision=hp)
                    + bc2_ref[...])


_BN = 2000

_head_call = pl.pallas_call(
    _head_body,
    grid=(N // _BN,),
    in_specs=[
        pl.BlockSpec((2, _BN, D), lambda i: (0, i, 0)),
        pl.BlockSpec((2, _BN, CW), lambda i: (0, i, 0)),
        pl.BlockSpec((_BN, D), lambda i: (i, 0)),
        pl.BlockSpec((H, D), lambda i: (0, 0)),
        pl.BlockSpec((1, H), lambda i: (0, 0)),
        pl.BlockSpec((H, D), lambda i: (0, 0)),
        pl.BlockSpec((16, H), lambda i: (0, 0)),
        pl.BlockSpec((1, 16), lambda i: (0, 0)),
        pl.BlockSpec((2, 16), lambda i: (0, 0)),
        pl.BlockSpec((1, 2), lambda i: (0, 0)),
    ],
    out_specs=pl.BlockSpec((_BN, 2), lambda i: (i, 0)),
    out_shape=jax.ShapeDtypeStruct((N, 2), jnp.float32),
)


def kernel(x, edge_index, edge_weight, edge_features, adj, T,
           bn_weight, bn_bias, W_l, b_l, W_r, Wc1, bc1, Wc2, bc2):
    xn = _bn_call(x, bn_weight.reshape(1, D), bn_bias.reshape(1, D))
    zeros = jnp.zeros((ACC_ROWS, D), jnp.float32)
    zeros16 = jnp.zeros((ACC_ROWS, CW), jnp.float32)
    ones = jnp.ones((G, CW), jnp.float32)
    acc, cnts = _sc_scatter(xn, edge_index, zeros, zeros16, ones)
    out = _head_call(acc, cnts, xn, W_l, b_l.reshape(1, H), W_r,
                     Wc1, bc1.reshape(1, 16), Wc2, bc2.reshape(1, 2))
    return out
